# baseline (speedup 1.0000x reference)
import jax
import jax.numpy as jnp
from jax.experimental import pallas as pl
from jax.experimental.pallas import tpu as pltpu

_THRESHOLD = 0.5
_NEG_POS_RATIO = 3
_ALPHA = 1.0


def _mbl_kernel(locs_ref, scores_ref, boxes_ref, labels_ref, priors_ref,
                locabs_ref, confpos_ref, confhard_ref, npos_ref,
                cneg_s, nposv_s):
    P = priors_ref.shape[1]
    NOBJ = boxes_ref.shape[1]
    f32 = jnp.float32
    i32 = jnp.int32

    priors = priors_ref[:]
    pcx = priors[0:1, :]
    pcy = priors[1:2, :]
    pw = priors[2:3, :]
    ph = priors[3:4, :]
    px1 = pcx - pw * 0.5
    py1 = pcy - ph * 0.5
    px2 = pcx + pw * 0.5
    py2 = pcy + ph * 0.5
    parea = pw * ph

    boxes_i = boxes_ref[0]
    labels_col = labels_ref[0].reshape(NOBJ, 1)
    p_iota = jax.lax.broadcasted_iota(i32, (NOBJ, P), 1)
    j_iota = jax.lax.broadcasted_iota(i32, (NOBJ, P), 0)

    bx1 = boxes_i[:, 0:1]
    by1 = boxes_i[:, 1:2]
    bx2 = boxes_i[:, 2:3]
    by2 = boxes_i[:, 3:4]
    barea = (bx2 - bx1) * (by2 - by1)

    iw = jnp.clip(jnp.minimum(bx2, px2) - jnp.maximum(bx1, px1), 0.0, None)
    ih = jnp.clip(jnp.minimum(by2, py2) - jnp.maximum(by1, py1), 0.0, None)
    inter = iw * ih
    ov = inter / (barea + parea - inter)

    ofp = jnp.max(ov, axis=0, keepdims=True)
    obj_fp = jnp.min(jnp.where(ov == ofp, j_iota, NOBJ),
                     axis=0, keepdims=True)
    m = jnp.max(ov, axis=1, keepdims=True)
    pfo = jnp.min(jnp.where(ov == m, p_iota, P),
                  axis=1, keepdims=True)
    match = p_iota == pfo
    jstar = jnp.max(jnp.where(match, j_iota, -1), axis=0, keepdims=True)
    forced = jstar >= 0
    obj_fp = jnp.where(forced, jstar, obj_fp)
    ofp = jnp.where(forced, 1.0, ofp)

    onehot_f = (obj_fp == j_iota).astype(f32)
    cx_o = (bx1 + bx2) * 0.5
    cy_o = (by1 + by2) * 0.5
    w_o = jnp.maximum(bx2 - bx1, 1e-20)
    h_o = jnp.maximum(by2 - by1, 1e-20)
    table = jnp.concatenate(
        [labels_col.astype(f32), cx_o, cy_o, w_o, h_o], axis=1).T
    gathered = jax.lax.dot_general(
        table, onehot_f, (((1,), (0,)), ((), ())),
        precision=jax.lax.Precision.HIGHEST,
        preferred_element_type=f32)
    lab = gathered[0:1, :].astype(i32)
    cx = gathered[1:2, :]
    cy = gathered[2:3, :]
    w = gathered[3:4, :]
    h = gathered[4:5, :]

    tc = jnp.where(ofp < _THRESHOLD, 0, lab)

    tl0 = (cx - pcx) / (pw * 0.1)
    tl1 = (cy - pcy) / (ph * 0.1)
    tl2 = jnp.log(w / pw) * 5.0
    tl3 = jnp.log(h / ph) * 5.0

    pos = tc > 0
    posf = pos.astype(f32)
    n_pos = jnp.sum(pos.astype(i32))

    locs = locs_ref[0]
    la = (jnp.abs(locs[0:1, :] - tl0)
          + jnp.abs(locs[1:2, :] - tl1)
          + jnp.abs(locs[2:3, :] - tl2)
          + jnp.abs(locs[3:4, :] - tl3))
    loc_abs_i = jnp.sum(la * posf)

    sc = scores_ref[0]
    l0 = sc[0:1, :]
    l1 = sc[1:2, :]
    l2 = sc[2:3, :]
    t0 = (tc == 0).astype(f32)
    t1 = ((tc == 1) | (tc == 3)).astype(f32)
    t2 = ((tc == 2) | (tc == 3)).astype(f32)

    def bce(l, t):
        return jnp.maximum(l, 0.0) - l * t + jnp.log1p(jnp.exp(-jnp.abs(l)))

    conf = bce(l0, t0) + bce(l1, t1) + bce(l2, t2)
    conf_pos_i = jnp.sum(conf * posf)
    cneg = jnp.where(pos, 0.0, conf)

    i = pl.program_id(0)
    B = pl.num_programs(0)
    cneg_s[pl.ds(i, 1), :] = cneg
    nposv_s[pl.ds(i, 1), :] = n_pos.reshape(1, 1)

    locabs_ref[...] = loc_abs_i.reshape(1, 1, 1)
    confpos_ref[...] = conf_pos_i.reshape(1, 1, 1)
    npos_ref[...] = n_pos.reshape(1, 1, 1)

    @pl.when(i == B - 1)
    def _():
        cn = cneg_s[:, :]
        bits = jax.lax.bitcast_convert_type(cn, i32)
        kv = nposv_s[:, :] * _NEG_POS_RATIO

        def search_body(b, t):
            t_try = t | jax.lax.shift_left(i32(1), i32(30) - b)
            cnt = jnp.sum((bits >= t_try).astype(i32), axis=1, keepdims=True)
            return jnp.where(cnt >= kv, t_try, t)

        thr = jax.lax.fori_loop(
            0, 31, search_body, jnp.zeros(kv.shape, i32))
        gt = bits > thr
        cnt_gt = jnp.sum(gt.astype(i32), axis=1, keepdims=True)
        sum_gt = jnp.sum(jnp.where(gt, cn, 0.0), axis=1, keepdims=True)
        thr_f = jnp.max(jnp.where(bits == thr, cn, 0.0),
                        axis=1, keepdims=True)
        hard = jnp.where(kv > 0,
                         sum_gt + (kv - cnt_gt).astype(f32) * thr_f,
                         0.0)
        confhard_ref[...] = jnp.sum(hard).reshape(1, 1)


def kernel(predicted_locs, predicted_scores, boxes, labels, priors_cxcy):
    B, P, _ = predicted_locs.shape
    NC = predicted_scores.shape[2]
    NOBJ = boxes.shape[1]
    labels3 = labels.reshape(B, 1, NOBJ)
    locs_t = jnp.transpose(predicted_locs, (0, 2, 1))
    scores_t = jnp.transpose(predicted_scores, (0, 2, 1))
    priors_t = priors_cxcy.T

    out_shape = [
        jax.ShapeDtypeStruct((B, 1, 1), jnp.float32),
        jax.ShapeDtypeStruct((B, 1, 1), jnp.float32),
        jax.ShapeDtypeStruct((1, 1), jnp.float32),
        jax.ShapeDtypeStruct((B, 1, 1), jnp.int32),
    ]
    loc_abs_v, conf_pos_v, conf_hard_v, npos_v = pl.pallas_call(
        _mbl_kernel,
        grid=(B,),
        in_specs=[
            pl.BlockSpec((1, 4, P), lambda i: (i, 0, 0)),
            pl.BlockSpec((1, NC, P), lambda i: (i, 0, 0)),
            pl.BlockSpec((1, NOBJ, 4), lambda i: (i, 0, 0)),
            pl.BlockSpec((1, 1, NOBJ), lambda i: (i, 0, 0)),
            pl.BlockSpec((4, P), lambda i: (0, 0)),
        ],
        out_specs=[
            pl.BlockSpec((1, 1, 1), lambda i: (i, 0, 0)),
            pl.BlockSpec((1, 1, 1), lambda i: (i, 0, 0)),
            pl.BlockSpec((1, 1), lambda i: (0, 0)),
            pl.BlockSpec((1, 1, 1), lambda i: (i, 0, 0)),
        ],
        out_shape=out_shape,
        scratch_shapes=[
            pltpu.VMEM((B, P), jnp.float32),
            pltpu.VMEM((B, 1), jnp.int32),
        ],
        compiler_params=pltpu.CompilerParams(
            dimension_semantics=("arbitrary",)),
    )(locs_t, scores_t, boxes, labels3, priors_t)

    n_positives = npos_v.reshape(B)
    npt = n_positives.sum()
    nptf = npt.astype(jnp.float32)
    loc_loss = jnp.where(npt > 0,
                         loc_abs_v.sum() / (4.0 * jnp.maximum(nptf, 1.0)),
                         0.0)
    conf_loss = (conf_hard_v.sum() + conf_pos_v.sum()) / (1e-10 + nptf)
    return conf_loss + _ALPHA * loc_loss, conf_loss, loc_loss, n_positives


# speedup vs baseline: 1.0961x; 1.0961x over previous
import jax
import jax.numpy as jnp
from jax.experimental import pallas as pl
from jax.experimental.pallas import tpu as pltpu

_THRESHOLD = 0.5
_NEG_POS_RATIO = 3
_ALPHA = 1.0


def _mbl_kernel(locs_ref, scores_ref, boxes_ref, labels_ref, priors_ref,
                locabs_ref, confpos_ref, confhard_ref, npos_ref,
                cneg_s, nposv_s):
    P = priors_ref.shape[1]
    NOBJ = boxes_ref.shape[1]
    f32 = jnp.float32
    i32 = jnp.int32

    priors = priors_ref[:]
    pcx = priors[0:1, :]
    pcy = priors[1:2, :]
    pw = priors[2:3, :]
    ph = priors[3:4, :]
    px1 = pcx - pw * 0.5
    py1 = pcy - ph * 0.5
    px2 = pcx + pw * 0.5
    py2 = pcy + ph * 0.5
    parea = pw * ph

    boxes_i = boxes_ref[0]
    labels_col = labels_ref[0].reshape(NOBJ, 1)
    p_iota = jax.lax.broadcasted_iota(i32, (NOBJ, P), 1)
    j_iota = jax.lax.broadcasted_iota(i32, (NOBJ, P), 0)

    bx1 = boxes_i[:, 0:1]
    by1 = boxes_i[:, 1:2]
    bx2 = boxes_i[:, 2:3]
    by2 = boxes_i[:, 3:4]
    barea = (bx2 - bx1) * (by2 - by1)

    iw = jnp.clip(jnp.minimum(bx2, px2) - jnp.maximum(bx1, px1), 0.0, None)
    ih = jnp.clip(jnp.minimum(by2, py2) - jnp.maximum(by1, py1), 0.0, None)
    inter = iw * ih
    ov = inter / (barea + parea - inter)

    ofp = jnp.max(ov, axis=0, keepdims=True)
    obj_fp = jnp.min(jnp.where(ov == ofp, j_iota, NOBJ),
                     axis=0, keepdims=True)
    m = jnp.max(ov, axis=1, keepdims=True)
    pfo = jnp.min(jnp.where(ov == m, p_iota, P),
                  axis=1, keepdims=True)
    match = p_iota == pfo
    jstar = jnp.max(jnp.where(match, j_iota, -1), axis=0, keepdims=True)
    forced = jstar >= 0
    obj_fp = jnp.where(forced, jstar, obj_fp)
    ofp = jnp.where(forced, 1.0, ofp)

    onehot_f = (obj_fp == j_iota).astype(f32)
    cx_o = (bx1 + bx2) * 0.5
    cy_o = (by1 + by2) * 0.5
    w_o = jnp.maximum(bx2 - bx1, 1e-20)
    h_o = jnp.maximum(by2 - by1, 1e-20)
    table = jnp.concatenate(
        [labels_col.astype(f32), cx_o, cy_o, w_o, h_o], axis=1).T
    gathered = jax.lax.dot_general(
        table, onehot_f, (((1,), (0,)), ((), ())),
        preferred_element_type=f32)
    lab = gathered[0:1, :].astype(i32)
    cx = gathered[1:2, :]
    cy = gathered[2:3, :]
    w = gathered[3:4, :]
    h = gathered[4:5, :]

    tc = jnp.where(ofp < _THRESHOLD, 0, lab)

    tl0 = (cx - pcx) / (pw * 0.1)
    tl1 = (cy - pcy) / (ph * 0.1)
    tl2 = jnp.log(w / pw) * 5.0
    tl3 = jnp.log(h / ph) * 5.0

    pos = tc > 0
    posf = pos.astype(f32)
    n_pos = jnp.sum(pos.astype(i32))

    locs = locs_ref[0]
    la = (jnp.abs(locs[0:1, :] - tl0)
          + jnp.abs(locs[1:2, :] - tl1)
          + jnp.abs(locs[2:3, :] - tl2)
          + jnp.abs(locs[3:4, :] - tl3))
    loc_abs_i = jnp.sum(la * posf)

    sc = scores_ref[0]
    l0 = sc[0:1, :]
    l1 = sc[1:2, :]
    l2 = sc[2:3, :]
    t0 = (tc == 0).astype(f32)
    t1 = ((tc == 1) | (tc == 3)).astype(f32)
    t2 = ((tc == 2) | (tc == 3)).astype(f32)

    def bce(l, t):
        return jnp.maximum(l, 0.0) - l * t + jnp.log1p(jnp.exp(-jnp.abs(l)))

    conf = bce(l0, t0) + bce(l1, t1) + bce(l2, t2)
    conf_pos_i = jnp.sum(conf * posf)
    cneg = jnp.where(pos, 0.0, conf)

    i = pl.program_id(0)
    B = pl.num_programs(0)
    cneg_s[pl.ds(i, 1), :] = cneg
    nposv_s[pl.ds(i, 1), :] = n_pos.reshape(1, 1)

    locabs_ref[...] = loc_abs_i.reshape(1, 1, 1)
    confpos_ref[...] = conf_pos_i.reshape(1, 1, 1)
    npos_ref[...] = n_pos.reshape(1, 1, 1)

    @pl.when(i == B - 1)
    def _():
        cn = cneg_s[:, :]
        bits = jax.lax.bitcast_convert_type(cn, i32)
        kv = nposv_s[:, :] * _NEG_POS_RATIO

        def search_body(b, t):
            t_try = t | jax.lax.shift_left(i32(1), i32(30) - b)
            cnt = jnp.sum((bits >= t_try).astype(i32), axis=1, keepdims=True)
            return jnp.where(cnt >= kv, t_try, t)

        thr = jax.lax.fori_loop(
            0, 31, search_body, jnp.zeros(kv.shape, i32))
        gt = bits > thr
        cnt_gt = jnp.sum(gt.astype(i32), axis=1, keepdims=True)
        sum_gt = jnp.sum(jnp.where(gt, cn, 0.0), axis=1, keepdims=True)
        thr_f = jnp.max(jnp.where(bits == thr, cn, 0.0),
                        axis=1, keepdims=True)
        hard = jnp.where(kv > 0,
                         sum_gt + (kv - cnt_gt).astype(f32) * thr_f,
                         0.0)
        confhard_ref[...] = jnp.sum(hard).reshape(1, 1)


def kernel(predicted_locs, predicted_scores, boxes, labels, priors_cxcy):
    B, P, _ = predicted_locs.shape
    NC = predicted_scores.shape[2]
    NOBJ = boxes.shape[1]
    labels3 = labels.reshape(B, 1, NOBJ)
    locs_t = jnp.transpose(predicted_locs, (0, 2, 1))
    scores_t = jnp.transpose(predicted_scores, (0, 2, 1))
    priors_t = priors_cxcy.T

    out_shape = [
        jax.ShapeDtypeStruct((B, 1, 1), jnp.float32),
        jax.ShapeDtypeStruct((B, 1, 1), jnp.float32),
        jax.ShapeDtypeStruct((1, 1), jnp.float32),
        jax.ShapeDtypeStruct((B, 1, 1), jnp.int32),
    ]
    loc_abs_v, conf_pos_v, conf_hard_v, npos_v = pl.pallas_call(
        _mbl_kernel,
        grid=(B,),
        in_specs=[
            pl.BlockSpec((1, 4, P), lambda i: (i, 0, 0)),
            pl.BlockSpec((1, NC, P), lambda i: (i, 0, 0)),
            pl.BlockSpec((1, NOBJ, 4), lambda i: (i, 0, 0)),
            pl.BlockSpec((1, 1, NOBJ), lambda i: (i, 0, 0)),
            pl.BlockSpec((4, P), lambda i: (0, 0)),
        ],
        out_specs=[
            pl.BlockSpec((1, 1, 1), lambda i: (i, 0, 0)),
            pl.BlockSpec((1, 1, 1), lambda i: (i, 0, 0)),
            pl.BlockSpec((1, 1), lambda i: (0, 0)),
            pl.BlockSpec((1, 1, 1), lambda i: (i, 0, 0)),
        ],
        out_shape=out_shape,
        scratch_shapes=[
            pltpu.VMEM((B, P), jnp.float32),
            pltpu.VMEM((B, 1), jnp.int32),
        ],
        compiler_params=pltpu.CompilerParams(
            dimension_semantics=("arbitrary",)),
    )(locs_t, scores_t, boxes, labels3, priors_t)

    n_positives = npos_v.reshape(B)
    npt = n_positives.sum()
    nptf = npt.astype(jnp.float32)
    loc_loss = jnp.where(npt > 0,
                         loc_abs_v.sum() / (4.0 * jnp.maximum(nptf, 1.0)),
                         0.0)
    conf_loss = (conf_hard_v.sum() + conf_pos_v.sum()) / (1e-10 + nptf)
    return conf_loss + _ALPHA * loc_loss, conf_loss, loc_loss, n_positives
